# Initial kernel scaffold; baseline (speedup 1.0000x reference)
import jax
import jax.numpy as jnp
from jax.experimental import pallas as pl


def kernel(hidden_states, expert_affinities, expert_index, w_gate, w_up, w_down):
    raise NotImplementedError("write your pallas kernel here")


# speedup vs baseline: 1.0457x; 1.0457x over previous
import jax
import jax.numpy as jnp
from jax.experimental import pallas as pl
from jax.experimental.pallas import tpu as pltpu

NUM_EXPERTS = 8
TOP_K = 2
HIDDEN = 1024
INTERMEDIATE = 2048
T = 2048

_SUB = 512


def _fused_moe_kernel(x_ref, aff_ref, idx_ref, wg_ref, wu_ref, wd_ref, out_ref):
    e = pl.program_id(0)

    idx0 = idx_ref[:, 0:1]
    idx1 = idx_ref[:, 1:2]
    iota = jax.lax.broadcasted_iota(jnp.int32, (T, NUM_EXPERTS), 1)
    mask = (idx0 == iota) | (idx1 == iota)
    aff = aff_ref[...]
    masked = jnp.where(mask, aff, 0.0)
    denom = jnp.clip(jnp.sum(jnp.abs(masked), axis=1, keepdims=True), 1e-12, None)
    w_e = masked[:, :] / denom
    sel = jnp.sum(jnp.where(iota == e, w_e, 0.0), axis=1, keepdims=True)

    wg = wg_ref[0]
    wu = wu_ref[0]
    wd = wd_ref[0]

    for s in range(T // _SUB):
        xs = x_ref[pl.ds(s * _SUB, _SUB), :]
        g = jnp.dot(xs, wg, preferred_element_type=jnp.float32)
        u = jnp.dot(xs, wu, preferred_element_type=jnp.float32)
        inter = (jax.nn.silu(g) * u).astype(jnp.bfloat16)
        y = jnp.dot(inter, wd, preferred_element_type=jnp.float32)
        y = y * sel[s * _SUB:(s + 1) * _SUB, :]

        @pl.when(e == 0)
        def _():
            out_ref[pl.ds(s * _SUB, _SUB), :] = y

        @pl.when(e != 0)
        def _():
            out_ref[pl.ds(s * _SUB, _SUB), :] += y


def kernel(hidden_states, expert_affinities, expert_index, w_gate, w_up, w_down):
    x16 = hidden_states.astype(jnp.bfloat16)
    wg16 = w_gate.astype(jnp.bfloat16)
    wu16 = w_up.astype(jnp.bfloat16)
    wd16 = w_down.astype(jnp.bfloat16)

    return pl.pallas_call(
        _fused_moe_kernel,
        grid=(NUM_EXPERTS,),
        in_specs=[
            pl.BlockSpec((T, HIDDEN), lambda e: (0, 0)),
            pl.BlockSpec((T, NUM_EXPERTS), lambda e: (0, 0)),
            pl.BlockSpec((T, TOP_K), lambda e: (0, 0)),
            pl.BlockSpec((1, HIDDEN, INTERMEDIATE), lambda e: (e, 0, 0)),
            pl.BlockSpec((1, HIDDEN, INTERMEDIATE), lambda e: (e, 0, 0)),
            pl.BlockSpec((1, INTERMEDIATE, HIDDEN), lambda e: (e, 0, 0)),
        ],
        out_specs=pl.BlockSpec((T, HIDDEN), lambda e: (0, 0)),
        out_shape=jax.ShapeDtypeStruct((T, HIDDEN), jnp.float32),
        compiler_params=pltpu.CompilerParams(
            dimension_semantics=("arbitrary",),
        ),
    )(x16, expert_affinities, expert_index, wg16, wu16, wd16)
